# Initial kernel scaffold; baseline (speedup 1.0000x reference)
import jax
import jax.numpy as jnp
from jax.experimental import pallas as pl


def kernel(features, rois):
    raise NotImplementedError("write your pallas kernel here")


# speedup vs baseline: 9.4572x; 9.4572x over previous
import functools

import jax
import jax.numpy as jnp
from jax import lax
from jax.experimental import pallas as pl
from jax.experimental.pallas import tpu as pltpu
from jax.experimental.pallas import tpu_sc as plsc

_POOLED = 7
_GROUP = 7
_OUT_DIM = 10
_SCALE = 0.125

_B = 2
_C = _OUT_DIM * _GROUP * _GROUP
_H = 64
_W = 64
_N = 512

_PQ = _GROUP * _GROUP
_DP = 16
_HP = _H + 1
_WP = _W + 1
_TBL_BLOCKS = _B * _PQ
_TBL_ROWS = _TBL_BLOCKS * _HP * _WP

_NC = 2
_NS = 16
_NW = _NC * _NS
_ITEMS = _N * _PQ
_PER_TILE = _ITEMS // _NW
_CH = 7
_CW = _PER_TILE // _CH


def _integral_body(x_ref, o_ref):
    x = x_ref[0].reshape(_H, _W * _DP)
    for s in (1, 2, 4, 8, 16, 32):
        x = x + jnp.concatenate(
            [jnp.zeros((s, _W * _DP), jnp.float32), x[:-s, :]], axis=0)
    for s in (1, 2, 4, 8, 16, 32):
        sl = s * _DP
        x = x + jnp.concatenate(
            [jnp.zeros((_H, sl), jnp.float32), x[:, :-sl]], axis=1)
    full = jnp.concatenate([jnp.zeros((_H, _DP), jnp.float32), x], axis=1)
    full = jnp.concatenate(
        [jnp.zeros((1, _WP * _DP), jnp.float32), full], axis=0)
    o_ref[0] = full.reshape(_HP, _WP, _DP)


def _build_table(feats_p):
    return pl.pallas_call(
        _integral_body,
        grid=(_TBL_BLOCKS,),
        in_specs=[pl.BlockSpec((1, _H, _W, _DP), lambda i: (i, 0, 0, 0))],
        out_specs=pl.BlockSpec((1, _HP, _WP, _DP), lambda i: (i, 0, 0, 0)),
        out_shape=jax.ShapeDtypeStruct((_TBL_BLOCKS, _HP, _WP, _DP),
                                       jnp.float32),
    )(feats_p)


def _build_indices(rois):
    b = rois[:, 0].astype(jnp.int32)
    x1 = jnp.round(rois[:, 1]) * _SCALE
    y1 = jnp.round(rois[:, 2]) * _SCALE
    x2 = (jnp.round(rois[:, 3]) + 1.0) * _SCALE
    y2 = (jnp.round(rois[:, 4]) + 1.0) * _SCALE
    roi_w = jnp.maximum(x2 - x1, 0.1)
    roi_h = jnp.maximum(y2 - y1, 0.1)
    bin_h = roi_h / float(_POOLED)
    bin_w = roi_w / float(_POOLED)
    pidx = jnp.arange(_POOLED, dtype=jnp.float32)
    hs = jnp.clip(jnp.floor(pidx[None, :] * bin_h[:, None] + y1[:, None]),
                  0, _H).astype(jnp.int32)
    he = jnp.clip(jnp.ceil((pidx[None, :] + 1.0) * bin_h[:, None]
                           + y1[:, None]), 0, _H).astype(jnp.int32)
    ws = jnp.clip(jnp.floor(pidx[None, :] * bin_w[:, None] + x1[:, None]),
                  0, _W).astype(jnp.int32)
    we = jnp.clip(jnp.ceil((pidx[None, :] + 1.0) * bin_w[:, None]
                           + x1[:, None]), 0, _W).astype(jnp.int32)

    hs2 = jnp.broadcast_to(hs[:, :, None], (_N, _POOLED, _POOLED))
    he2 = jnp.broadcast_to(he[:, :, None], (_N, _POOLED, _POOLED))
    ws2 = jnp.broadcast_to(ws[:, None, :], (_N, _POOLED, _POOLED))
    we2 = jnp.broadcast_to(we[:, None, :], (_N, _POOLED, _POOLED))
    pq = jnp.arange(_PQ, dtype=jnp.int32).reshape(1, _POOLED, _POOLED)
    base = (b[:, None, None] * _PQ + pq) * (_HP * _WP)
    idx = jnp.stack([
        base + he2 * _WP + we2,
        base + hs2 * _WP + we2,
        base + he2 * _WP + ws2,
        base + hs2 * _WP + ws2,
    ]).reshape(4, _N, _PQ)
    cnt = ((he2 - hs2) * (we2 - ws2)).astype(jnp.float32).reshape(_N, _PQ)
    recip = jnp.where(cnt > 0.0, 1.0 / jnp.maximum(cnt, 1.0), 0.0)
    return idx, recip


def _sc_pool_body(table_ref, idx_ref, recip_ref, out_ref,
                  idx_v, recip_v, r0, r1, r2, r3, out_v, sem):
    wid = lax.axis_index("s") * _NC + lax.axis_index("c")
    for k in range(4):
        pltpu.sync_copy(idx_ref.at[k, wid], idx_v.at[k])
    pltpu.sync_copy(recip_ref.at[wid], recip_v)
    rows = (r0, r1, r2, r3)
    copies = []
    for k in range(4):
        for j in range(_CH):
            copies.append(
                pltpu.async_copy(table_ref.at[idx_v.at[k, j]],
                                 rows[k].at[pl.ds(j * _CW, _CW)], sem))
    for cp in copies:
        cp.wait()

    def body(g, _):
        base = g * 16
        rv = recip_v[pl.ds(base, 16)]
        for t in range(16):
            i = base + t
            out_v[i] = ((r0[i] - r1[i]) - (r2[i] - r3[i])) * rv[t]
        return 0
    lax.fori_loop(0, _PER_TILE // 16, body, 0)
    pltpu.sync_copy(out_v, out_ref.at[wid])


def _sc_pool(table_flat, idx_r, recip_r):
    mesh = plsc.VectorSubcoreMesh(core_axis_name="c", subcore_axis_name="s",
                                  num_cores=_NC, num_subcores=_NS)
    fn = pl.kernel(
        _sc_pool_body,
        out_type=jax.ShapeDtypeStruct((_NW, _PER_TILE, _DP), jnp.float32),
        mesh=mesh,
        compiler_params=pltpu.CompilerParams(use_tc_tiling_on_sc=False),
        scratch_types=[
            pltpu.VMEM((4, _CH, _CW), jnp.int32),
            pltpu.VMEM((_PER_TILE,), jnp.float32),
            pltpu.VMEM((_PER_TILE, _DP), jnp.float32),
            pltpu.VMEM((_PER_TILE, _DP), jnp.float32),
            pltpu.VMEM((_PER_TILE, _DP), jnp.float32),
            pltpu.VMEM((_PER_TILE, _DP), jnp.float32),
            pltpu.VMEM((_PER_TILE, _DP), jnp.float32),
            pltpu.SemaphoreType.DMA,
        ],
    )
    return fn(table_flat, idx_r, recip_r)


@jax.jit
def kernel(features, rois):
    feats_r = features.reshape(_B, _OUT_DIM, _PQ, _H, _W)
    feats_t = jnp.transpose(feats_r, (0, 2, 3, 4, 1))
    feats_p = jnp.concatenate(
        [feats_t,
         jnp.zeros((_B, _PQ, _H, _W, _DP - _OUT_DIM), jnp.float32)],
        axis=-1).reshape(_TBL_BLOCKS, _H, _W, _DP)

    table = _build_table(feats_p).reshape(_TBL_ROWS, _DP)
    idx, recip = _build_indices(rois)
    idx_r = idx.reshape(4, _NW, _CH, _CW)
    recip_r = recip.reshape(_NW, _PER_TILE)

    pooled = _sc_pool(table, idx_r, recip_r)
    out = pooled.reshape(_N, _PQ, _DP)[:, :, :_OUT_DIM]
    out = out.reshape(_N, _POOLED, _POOLED, _OUT_DIM)
    return jnp.transpose(out, (0, 3, 1, 2))
